# Initial kernel scaffold; baseline (speedup 1.0000x reference)
import jax
import jax.numpy as jnp
from jax.experimental import pallas as pl


def kernel(index, table):
    raise NotImplementedError("write your pallas kernel here")


# speedup vs baseline: 4.0839x; 4.0839x over previous
import functools

import jax
import jax.numpy as jnp
from jax import lax
from jax.experimental import pallas as pl
from jax.experimental.pallas import tpu as pltpu
from jax.experimental.pallas import tpu_sc as plsc

EMBED_DIM = 64
NUM_CORES = 2
NUM_SUBCORES = 16
NUM_WORKERS = NUM_CORES * NUM_SUBCORES
CHUNK = 512
IDX_PER_GATHER = 128


@functools.lru_cache(maxsize=None)
def _make_kernel(num_rows: int, vocab: int):
    rows_per_worker = num_rows // NUM_WORKERS
    n_chunks = rows_per_worker // CHUNK
    assert rows_per_worker % CHUNK == 0
    mesh = plsc.VectorSubcoreMesh(
        core_axis_name="c", subcore_axis_name="s")

    @functools.partial(
        pl.kernel,
        mesh=mesh,
        out_type=jax.ShapeDtypeStruct((num_rows, EMBED_DIM), jnp.float32),
        scratch_types=[
            pltpu.VMEM((rows_per_worker,), jnp.int32),
            pltpu.VMEM((CHUNK, EMBED_DIM), jnp.float32),
            pltpu.SemaphoreType.DMA,
        ],
        compiler_params=pltpu.CompilerParams(use_tc_tiling_on_sc=False),
    )
    def emb_kernel(idx_hbm, table_hbm, out_hbm, idx_v, rows_v, gsem):
        wid = lax.axis_index("s") * NUM_CORES + lax.axis_index("c")
        base = wid * rows_per_worker
        pltpu.sync_copy(idx_hbm.at[pl.ds(base, rows_per_worker)], idx_v)

        @pl.loop(0, n_chunks)
        def _(ci):
            off = ci * CHUNK
            copies = []
            for j in range(CHUNK // IDX_PER_GATHER):
                copies.append(pltpu.async_copy(
                    table_hbm.at[
                        idx_v.at[pl.ds(off + j * IDX_PER_GATHER,
                                       IDX_PER_GATHER)]],
                    rows_v.at[pl.ds(j * IDX_PER_GATHER, IDX_PER_GATHER)],
                    gsem))
            for c in copies:
                c.wait()
            pltpu.sync_copy(rows_v, out_hbm.at[pl.ds(base + off, CHUNK)])

    return emb_kernel


def kernel(index, table):
    batch, hist = index.shape
    num_rows = batch * hist
    flat_idx = index.reshape(num_rows)
    out = _make_kernel(num_rows, table.shape[0])(flat_idx, table)
    return out.reshape(batch, hist, EMBED_DIM)
